# baseline (speedup 1.0000x reference)
import functools

import jax
import jax.numpy as jnp
from jax import lax
from jax.experimental import pallas as pl
from jax.experimental.pallas import tpu as pltpu
from jax.experimental.pallas import tpu_sc as plsc

_NUM_EMBEDDINGS = 1024
_DIM = 64
_B = 9216

_NC, _NS = 2, 16
_NW = _NC * _NS
_BPW = _B // _NW
_NCH = 3
_CH = _BPW // _NCH


def _argmin_body(xt_ref, d_ref, idx_ref, tab_ref):
    xb = xt_ref[0].T
    dm = d_ref[...]
    sim = jnp.dot(xb, dm, preferred_element_type=jnp.float32)
    rn = jnp.sum(xb * xb, axis=1, keepdims=True)
    cn = jnp.sum(dm * dm, axis=0, keepdims=True)
    dist = rn + cn - 2.0 * sim
    minv = jnp.min(dist, axis=1, keepdims=True)
    iota = jax.lax.broadcasted_iota(jnp.int32, dist.shape, 1)
    masked = jnp.where(dist == minv, iota, _NUM_EMBEDDINGS)
    idx = jnp.min(masked, axis=1)
    idx_ref[...] = idx.reshape(2, _NCH, _CH)

    @pl.when(pl.program_id(0) == 0)
    def _():
        dt = dm.T
        tab_ref[...] = jnp.concatenate([dt, dt], axis=1)


def _code_indices(xt, dictionary):
    nb = xt.shape[0]
    return pl.pallas_call(
        _argmin_body,
        grid=(nb,),
        in_specs=[
            pl.BlockSpec((1, _DIM, 2 * _BPW), lambda i: (i, 0, 0)),
            pl.BlockSpec((_DIM, _NUM_EMBEDDINGS), lambda i: (0, 0)),
        ],
        out_specs=[
            pl.BlockSpec((2, _NCH, _CH), lambda i: (i, 0, 0)),
            pl.BlockSpec((_NUM_EMBEDDINGS, 2 * _DIM), lambda i: (0, 0)),
        ],
        out_shape=[
            jax.ShapeDtypeStruct((_NW, _NCH, _CH), jnp.int32),
            jax.ShapeDtypeStruct((_NUM_EMBEDDINGS, 2 * _DIM), jnp.float32),
        ],
    )(xt, dictionary)


@functools.cache
def _make_sc_gather():
    mesh = plsc.VectorSubcoreMesh(core_axis_name="c", subcore_axis_name="s")

    @functools.partial(
        pl.kernel,
        mesh=mesh,
        out_type=jax.ShapeDtypeStruct((_B // 576, 576, _DIM), jnp.float32),
        scratch_types=[
            pltpu.VMEM((_NCH, _CH), jnp.int32),
            pltpu.VMEM((_NCH, _CH, 2 * _DIM), jnp.float32),
            pltpu.VMEM((_NCH, _CH, _DIM), jnp.float32),
            pltpu.SemaphoreType.DMA,
            pltpu.SemaphoreType.DMA,
        ],
    )
    def _sc_gather(idx_hbm, table_hbm, out_hbm, idx_v, rows_v, out_v, sem, osem):
        wid = lax.axis_index("s") * _NC + lax.axis_index("c")
        b = wid // 2
        r0 = (wid % 2) * _BPW
        pltpu.sync_copy(idx_hbm.at[wid], idx_v)
        copies = [
            pltpu.async_copy(table_hbm.at[idx_v.at[k]], rows_v.at[k], sem)
            for k in range(_NCH)
        ]
        out_copies = []
        for k in range(_NCH):
            copies[k].wait()
            for p in range(_CH):
                for c in range(_DIM // 16):
                    out_v[k, p, pl.ds(c * 16, 16)] = rows_v[
                        k, p, pl.ds(c * 16, 16)
                    ]
            out_copies.append(
                pltpu.async_copy(
                    out_v.at[k],
                    out_hbm.at[b, pl.ds(r0 + k * _CH, _CH)],
                    osem,
                )
            )
        for cp in out_copies:
            cp.wait()

    return _sc_gather


def kernel(x, dictionary):
    orig_shape = x.shape
    xt = jnp.transpose(x, (0, 2, 1))
    idx, table = _code_indices(xt, dictionary)
    q = _make_sc_gather()(idx, table)
    return q.reshape(orig_shape)


# speedup vs baseline: 1.0360x; 1.0360x over previous
import functools

import jax
import jax.numpy as jnp
from jax import lax
from jax.experimental import pallas as pl
from jax.experimental.pallas import tpu as pltpu
from jax.experimental.pallas import tpu_sc as plsc

_NUM_EMBEDDINGS = 1024
_DIM = 64
_B = 9216

_NC, _NS = 2, 16
_NW = _NC * _NS
_BPW = _B // _NW
_NCH = 3
_CH = _BPW // _NCH


def _argmin_body(xt_ref, d_ref, idx_ref, tab_ref):
    xb = jnp.concatenate([xt_ref[0].T, xt_ref[1].T], axis=0)
    dm = d_ref[...]
    sim = jnp.dot(xb, dm, preferred_element_type=jnp.float32)
    rn = jnp.sum(xb * xb, axis=1, keepdims=True)
    cn = jnp.sum(dm * dm, axis=0, keepdims=True)
    dist = rn + cn - 2.0 * sim
    minv = jnp.min(dist, axis=1, keepdims=True)
    iota = jax.lax.broadcasted_iota(jnp.int32, dist.shape, 1)
    masked = jnp.where(dist == minv, iota, _NUM_EMBEDDINGS)
    idx = jnp.min(masked, axis=1)
    idx_ref[...] = idx.reshape(4, _NCH, _CH)

    @pl.when(pl.program_id(0) == 0)
    def _():
        dt = dm.T
        tab_ref[...] = jnp.concatenate([dt, dt], axis=1)


def _code_indices(xt, dictionary):
    nb = xt.shape[0]
    return pl.pallas_call(
        _argmin_body,
        grid=(nb // 2,),
        in_specs=[
            pl.BlockSpec((2, _DIM, 2 * _BPW), lambda i: (i, 0, 0)),
            pl.BlockSpec((_DIM, _NUM_EMBEDDINGS), lambda i: (0, 0)),
        ],
        out_specs=[
            pl.BlockSpec((4, _NCH, _CH), lambda i: (i, 0, 0)),
            pl.BlockSpec((_NUM_EMBEDDINGS, 2 * _DIM), lambda i: (0, 0)),
        ],
        out_shape=[
            jax.ShapeDtypeStruct((_NW, _NCH, _CH), jnp.int32),
            jax.ShapeDtypeStruct((_NUM_EMBEDDINGS, 2 * _DIM), jnp.float32),
        ],
    )(xt, dictionary)


@functools.cache
def _make_sc_gather():
    mesh = plsc.VectorSubcoreMesh(core_axis_name="c", subcore_axis_name="s")

    @functools.partial(
        pl.kernel,
        mesh=mesh,
        out_type=jax.ShapeDtypeStruct((_B // 576, 576, _DIM), jnp.float32),
        scratch_types=[
            pltpu.VMEM((_NCH, _CH), jnp.int32),
            pltpu.VMEM((_NCH, _CH, 2 * _DIM), jnp.float32),
            pltpu.VMEM((_NCH, _CH, _DIM), jnp.float32),
            pltpu.SemaphoreType.DMA,
            pltpu.SemaphoreType.DMA,
        ],
    )
    def _sc_gather(idx_hbm, table_hbm, out_hbm, idx_v, rows_v, out_v, sem, osem):
        wid = lax.axis_index("s") * _NC + lax.axis_index("c")
        b = wid // 2
        r0 = (wid % 2) * _BPW
        pltpu.sync_copy(idx_hbm.at[wid], idx_v)
        copies = [
            pltpu.async_copy(table_hbm.at[idx_v.at[k]], rows_v.at[k], sem)
            for k in range(_NCH)
        ]
        out_copies = []
        for k in range(_NCH):
            copies[k].wait()
            for p in range(_CH):
                for c in range(_DIM // 16):
                    out_v[k, p, pl.ds(c * 16, 16)] = rows_v[
                        k, p, pl.ds(c * 16, 16)
                    ]
            out_copies.append(
                pltpu.async_copy(
                    out_v.at[k],
                    out_hbm.at[b, pl.ds(r0 + k * _CH, _CH)],
                    osem,
                )
            )
        for cp in out_copies:
            cp.wait()

    return _sc_gather


def kernel(x, dictionary):
    orig_shape = x.shape
    xt = jnp.transpose(x, (0, 2, 1))
    idx, table = _code_indices(xt, dictionary)
    q = _make_sc_gather()(idx, table)
    return q.reshape(orig_shape)


# speedup vs baseline: 1.0452x; 1.0089x over previous
import functools

import jax
import jax.numpy as jnp
from jax import lax
from jax.experimental import pallas as pl
from jax.experimental.pallas import tpu as pltpu
from jax.experimental.pallas import tpu_sc as plsc

_NUM_EMBEDDINGS = 1024
_DIM = 64
_B = 9216

_NC, _NS = 2, 16
_NW = _NC * _NS
_BPW = _B // _NW
_NCH = 3
_CH = _BPW // _NCH


def _argmin_body(xt_ref, d_ref, idx_ref, tab_ref):
    xb = jnp.concatenate([xt_ref[i].T for i in range(4)], axis=0)
    dm = d_ref[...]
    sim = jnp.dot(xb, dm, preferred_element_type=jnp.float32)
    rn = jnp.sum(xb * xb, axis=1, keepdims=True)
    cn = jnp.sum(dm * dm, axis=0, keepdims=True)
    dist = rn + cn - 2.0 * sim
    minv = jnp.min(dist, axis=1, keepdims=True)
    iota = jax.lax.broadcasted_iota(jnp.int32, dist.shape, 1)
    masked = jnp.where(dist == minv, iota, _NUM_EMBEDDINGS)
    idx = jnp.min(masked, axis=1)
    idx_ref[...] = idx.reshape(8, _NCH, _CH)

    @pl.when(pl.program_id(0) == 0)
    def _():
        dt = dm.T
        tab_ref[...] = jnp.concatenate([dt, dt], axis=1)


def _code_indices(xt, dictionary):
    nb = xt.shape[0]
    return pl.pallas_call(
        _argmin_body,
        grid=(nb // 4,),
        in_specs=[
            pl.BlockSpec((4, _DIM, 2 * _BPW), lambda i: (i, 0, 0)),
            pl.BlockSpec((_DIM, _NUM_EMBEDDINGS), lambda i: (0, 0)),
        ],
        out_specs=[
            pl.BlockSpec((8, _NCH, _CH), lambda i: (i, 0, 0)),
            pl.BlockSpec((_NUM_EMBEDDINGS, 2 * _DIM), lambda i: (0, 0)),
        ],
        out_shape=[
            jax.ShapeDtypeStruct((_NW, _NCH, _CH), jnp.int32),
            jax.ShapeDtypeStruct((_NUM_EMBEDDINGS, 2 * _DIM), jnp.float32),
        ],
    )(xt, dictionary)


@functools.cache
def _make_sc_gather():
    mesh = plsc.VectorSubcoreMesh(core_axis_name="c", subcore_axis_name="s")

    @functools.partial(
        pl.kernel,
        mesh=mesh,
        out_type=jax.ShapeDtypeStruct((_B // 576, 576, _DIM), jnp.float32),
        scratch_types=[
            pltpu.VMEM((_NCH, _CH), jnp.int32),
            pltpu.VMEM((_NCH, _CH, 2 * _DIM), jnp.float32),
            pltpu.VMEM((_NCH, _CH, _DIM), jnp.float32),
            pltpu.SemaphoreType.DMA,
            pltpu.SemaphoreType.DMA,
        ],
    )
    def _sc_gather(idx_hbm, table_hbm, out_hbm, idx_v, rows_v, out_v, sem, osem):
        wid = lax.axis_index("s") * _NC + lax.axis_index("c")
        b = wid // 2
        r0 = (wid % 2) * _BPW
        pltpu.sync_copy(idx_hbm.at[wid], idx_v)
        copies = [
            pltpu.async_copy(table_hbm.at[idx_v.at[k]], rows_v.at[k], sem)
            for k in range(_NCH)
        ]
        out_copies = []
        for k in range(_NCH):
            copies[k].wait()
            for p in range(_CH):
                for c in range(_DIM // 16):
                    out_v[k, p, pl.ds(c * 16, 16)] = rows_v[
                        k, p, pl.ds(c * 16, 16)
                    ]
            out_copies.append(
                pltpu.async_copy(
                    out_v.at[k],
                    out_hbm.at[b, pl.ds(r0 + k * _CH, _CH)],
                    osem,
                )
            )
        for cp in out_copies:
            cp.wait()

    return _sc_gather


def kernel(x, dictionary):
    orig_shape = x.shape
    xt = jnp.transpose(x, (0, 2, 1))
    idx, table = _code_indices(xt, dictionary)
    q = _make_sc_gather()(idx, table)
    return q.reshape(orig_shape)


# speedup vs baseline: 1.0887x; 1.0416x over previous
import functools

import jax
import jax.numpy as jnp
from jax import lax
from jax.experimental import pallas as pl
from jax.experimental.pallas import tpu as pltpu
from jax.experimental.pallas import tpu_sc as plsc

_NUM_EMBEDDINGS = 1024
_DIM = 64
_B = 9216

_NC, _NS = 2, 16
_NW = _NC * _NS
_BPW = _B // _NW
_NCH = 3
_CH = _BPW // _NCH


def _argmin_body(xt_ref, d_ref, idx_ref, tab_ref):
    xb = jnp.concatenate([xt_ref[i].T for i in range(4)], axis=0)
    dm = d_ref[...]
    sim = jnp.dot(xb, dm, preferred_element_type=jnp.float32)
    rn = jnp.sum(xb * xb, axis=1, keepdims=True)
    cn = jnp.sum(dm * dm, axis=0, keepdims=True)
    dist = rn + cn - 2.0 * sim
    minv = jnp.min(dist, axis=1, keepdims=True)
    iota = jax.lax.broadcasted_iota(jnp.int32, dist.shape, 1)
    masked = jnp.where(dist == minv, iota, _NUM_EMBEDDINGS)
    idx = jnp.min(masked, axis=1)
    idx_ref[...] = idx.reshape(8, _NCH, _CH)

    @pl.when(pl.program_id(0) == 0)
    def _():
        dt = dm.T
        tab_ref[...] = jnp.concatenate([dt, dt], axis=1)


def _code_indices(xt, dictionary):
    nb = xt.shape[0]
    return pl.pallas_call(
        _argmin_body,
        grid=(nb // 4,),
        in_specs=[
            pl.BlockSpec((4, _DIM, 2 * _BPW), lambda i: (i, 0, 0)),
            pl.BlockSpec((_DIM, _NUM_EMBEDDINGS), lambda i: (0, 0)),
        ],
        out_specs=[
            pl.BlockSpec((8, _NCH, _CH), lambda i: (i, 0, 0)),
            pl.BlockSpec((_NUM_EMBEDDINGS, 2 * _DIM), lambda i: (0, 0)),
        ],
        out_shape=[
            jax.ShapeDtypeStruct((_NW, _NCH, _CH), jnp.int32),
            jax.ShapeDtypeStruct((_NUM_EMBEDDINGS, 2 * _DIM), jnp.float32),
        ],
    )(xt, dictionary)


@functools.cache
def _make_sc_gather():
    mesh = plsc.VectorSubcoreMesh(core_axis_name="c", subcore_axis_name="s")

    @functools.partial(
        pl.kernel,
        mesh=mesh,
        out_type=jax.ShapeDtypeStruct((_NW, _NCH, _CH, 2 * _DIM), jnp.float32),
        scratch_types=[
            pltpu.VMEM((_NCH, _CH), jnp.int32),
            pltpu.VMEM((_NCH, _CH, 2 * _DIM), jnp.float32),
            pltpu.SemaphoreType.DMA,
            pltpu.SemaphoreType.DMA,
        ],
    )
    def _sc_gather(idx_hbm, table_hbm, out_hbm, idx_v, rows_v, sem, osem):
        wid = lax.axis_index("s") * _NC + lax.axis_index("c")
        pltpu.sync_copy(idx_hbm.at[wid], idx_v)
        copies = [
            pltpu.async_copy(table_hbm.at[idx_v.at[k]], rows_v.at[k], sem)
            for k in range(_NCH)
        ]
        out_copies = []
        for k in range(_NCH):
            copies[k].wait()
            out_copies.append(
                pltpu.async_copy(rows_v.at[k], out_hbm.at[wid, k], osem)
            )
        for cp in out_copies:
            cp.wait()

    return _sc_gather


def _compact_body(g_ref, out_ref):
    for j in range(g_ref.shape[0]):
        out_ref[j] = g_ref[j][:, :_DIM].T


def _compact_transpose(g):
    nb = g.shape[0]
    return pl.pallas_call(
        _compact_body,
        grid=(nb // 4,),
        in_specs=[pl.BlockSpec((4, 576, 2 * _DIM), lambda i: (i, 0, 0))],
        out_specs=pl.BlockSpec((4, _DIM, 576), lambda i: (i, 0, 0)),
        out_shape=jax.ShapeDtypeStruct((nb, _DIM, 576), jnp.float32),
    )(g)


def kernel(x, dictionary):
    orig_shape = x.shape
    xt = jnp.transpose(x, (0, 2, 1))
    del orig_shape
    idx, table = _code_indices(xt, dictionary)
    g = _make_sc_gather()(idx, table)
    qt = _compact_transpose(g.reshape(_B // 576, 576, 2 * _DIM))
    return jnp.transpose(qt, (0, 2, 1))
